# baseline (speedup 1.0000x reference)
import functools

import jax
import jax.numpy as jnp
import numpy as np
from jax import lax
from jax.experimental import pallas as pl
from jax.experimental.pallas import tpu as pltpu
from jax.experimental.pallas import tpu_sc as plsc

_SPATIAL_START = [45, 48, 87, 90]
_PATCH_SIZES = [3, 5, 7, 9]

_B, _T, _N, _D = 8, 16, 196, 768
_ROWS = _B * _T * _N
_NC, _NS = 2, 16
_SLAB = 4 * _N
_CHUNKS = [(0, 40), (40, 40), (80, 40), (120, 40), (160, 32), (192, 4)]


def _patch_indices(spatial_point, patch_size):
    sp = spatial_point - 15 * (_PATCH_SIZES.index(patch_size))
    gap = (patch_size + 1) // 2
    additional = [sp, sp + gap, sp + gap * 2,
                  sp + 14 * gap, sp + 14 * gap + gap * 2,
                  sp + 14 * gap * 2, sp + 14 * gap * 2 + gap,
                  sp + 14 * gap * 2 + gap * 2]
    center = [14 * i + sp + j + 1 for j in range(patch_size) for i in range(patch_size)]
    return np.asarray(sorted(additional + center), dtype=np.int64)


def _chunk_tables():
    slab_idx = []
    for sp in _SPATIAL_START:
        fr = np.concatenate([
            np.full(len(_patch_indices(sp, ps)), i, dtype=np.int64)
            for i, ps in enumerate(_PATCH_SIZES)])
        pid = np.concatenate([_patch_indices(sp, ps) for ps in _PATCH_SIZES])
        slab_idx.append(fr * _N + pid)
    slab_idx = np.stack(slab_idx).astype(np.int32)
    return [slab_idx[:, k0:k0 + ln].copy() for (k0, ln) in _CHUNKS]


_CHUNK_IDX = _chunk_tables()


@functools.cache
def _build_tube_gather():
    mesh = plsc.VectorSubcoreMesh(
        core_axis_name="c", subcore_axis_name="s",
        num_cores=_NC, num_subcores=_NS)

    idx_scratch = [pltpu.VMEM(a.shape, jnp.int32) for a in _CHUNK_IDX]
    buf_keys = ["A0", "A1", "A2", "A3", "C"]
    buf_scratch = [
        pltpu.VMEM((40, _D), jnp.float32),
        pltpu.VMEM((40, _D), jnp.float32),
        pltpu.VMEM((40, _D), jnp.float32),
        pltpu.VMEM((40, _D), jnp.float32),
        pltpu.VMEM((4, _D), jnp.float32),
    ]

    @functools.partial(
        pl.kernel,
        out_type=jax.ShapeDtypeStruct((_B, _T, _N, _D), jnp.float32),
        mesh=mesh,
        scratch_types=(idx_scratch + buf_scratch
                       + [pltpu.SemaphoreType.DMA] * (2 * len(buf_keys))),
    )
    def _tube_gather(x_hbm, *rest):
        nc = len(_CHUNK_IDX)
        idx_in = rest[:nc]
        out_hbm = rest[nc]
        sc = list(rest[nc + 1:])
        idx_v = sc[:nc]
        bufs = dict(zip(buf_keys, sc[nc:nc + len(buf_keys)]))
        sems = sc[nc + len(buf_keys):]
        gsem = dict(zip(buf_keys, sems[:len(buf_keys)]))
        ssem = dict(zip(buf_keys, sems[len(buf_keys):]))

        wid = lax.axis_index("s") * _NC + lax.axis_index("c")
        b = wid // 4
        blk = wid % 4
        t0 = 4 * blk
        slab0 = pl.multiple_of((16 * b + t0) * _N, _SLAB)
        slab = x_hbm.at[pl.ds(slab0, _SLAB)]

        for src, dst in zip(idx_in, idx_v):
            pltpu.sync_copy(src, dst)

        units = []
        for j in range(4):
            for u, (k0, ln) in enumerate(_CHUNKS):
                key = "C" if ln == 4 else f"A{len(units) % 4}"

                def g_u(key=key, j=j, u=u, ln=ln):
                    dst = (bufs[key] if ln in (4, 40)
                           else bufs[key].at[pl.ds(0, ln)])
                    return pltpu.async_copy(
                        slab.at[idx_v[u].at[j]], dst, gsem[key])

                def s_u(key=key, j=j, k0=k0, ln=ln):
                    src = (bufs[key] if ln in (4, 40)
                           else bufs[key].at[pl.ds(0, ln)])
                    return pltpu.async_copy(
                        src, out_hbm.at[b, t0 + j, pl.ds(k0, ln), :],
                        ssem[key])

                units.append((key, g_u, s_u))

        last_store = {}
        gathers = [None] * len(units)

        def issue(i):
            key = units[i][0]
            h = last_store.pop(key, None)
            if h is not None:
                h.wait()
            gathers[i] = units[i][1]()

        la = 3
        for i in range(la):
            issue(i)
        for i, (key, _, store) in enumerate(units):
            if i + la < len(units):
                issue(i + la)
            gathers[i].wait()
            last_store[key] = store()
        for h in last_store.values():
            h.wait()

    return _tube_gather


def kernel(x):
    args = [jnp.asarray(a) for a in _CHUNK_IDX]
    return _build_tube_gather()(x.reshape(_ROWS, _D), *args)


# speedup vs baseline: 1.0103x; 1.0103x over previous
import functools

import jax
import jax.numpy as jnp
import numpy as np
from jax import lax
from jax.experimental import pallas as pl
from jax.experimental.pallas import tpu as pltpu
from jax.experimental.pallas import tpu_sc as plsc

_SPATIAL_START = [45, 48, 87, 90]
_PATCH_SIZES = [3, 5, 7, 9]

_B, _T, _N, _D = 8, 16, 196, 768
_ROWS = _B * _T * _N
_NC, _NS = 2, 16
_SLAB = 4 * _N
_CHUNKS = [(0, 40), (40, 40), (80, 40), (120, 40), (160, 32), (192, 4)]


def _patch_indices(spatial_point, patch_size):
    sp = spatial_point - 15 * (_PATCH_SIZES.index(patch_size))
    gap = (patch_size + 1) // 2
    additional = [sp, sp + gap, sp + gap * 2,
                  sp + 14 * gap, sp + 14 * gap + gap * 2,
                  sp + 14 * gap * 2, sp + 14 * gap * 2 + gap,
                  sp + 14 * gap * 2 + gap * 2]
    center = [14 * i + sp + j + 1 for j in range(patch_size) for i in range(patch_size)]
    return np.asarray(sorted(additional + center), dtype=np.int64)


def _chunk_tables():
    slab_idx = []
    for sp in _SPATIAL_START:
        fr = np.concatenate([
            np.full(len(_patch_indices(sp, ps)), i, dtype=np.int64)
            for i, ps in enumerate(_PATCH_SIZES)])
        pid = np.concatenate([_patch_indices(sp, ps) for ps in _PATCH_SIZES])
        slab_idx.append(fr * _N + pid)
    slab_idx = np.stack(slab_idx).astype(np.int32)
    return [slab_idx[:, k0:k0 + ln].copy() for (k0, ln) in _CHUNKS]


_CHUNK_IDX = _chunk_tables()


@functools.cache
def _build_tube_gather():
    mesh = plsc.VectorSubcoreMesh(
        core_axis_name="c", subcore_axis_name="s",
        num_cores=_NC, num_subcores=_NS)

    idx_scratch = [pltpu.VMEM(a.shape, jnp.int32) for a in _CHUNK_IDX]
    buf_keys = ["A0", "A1", "A2", "A3", "C"]
    buf_scratch = [
        pltpu.VMEM((40, _D), jnp.float32),
        pltpu.VMEM((40, _D), jnp.float32),
        pltpu.VMEM((40, _D), jnp.float32),
        pltpu.VMEM((40, _D), jnp.float32),
        pltpu.VMEM((4, _D), jnp.float32),
    ]

    @functools.partial(
        pl.kernel,
        out_type=jax.ShapeDtypeStruct((_B, _T, _N, _D), jnp.float32),
        mesh=mesh,
        scratch_types=(idx_scratch + buf_scratch
                       + [pltpu.SemaphoreType.DMA] * (2 * len(buf_keys))),
    )
    def _tube_gather(x_hbm, *rest):
        nc = len(_CHUNK_IDX)
        idx_in = rest[:nc]
        out_hbm = rest[nc]
        sc = list(rest[nc + 1:])
        idx_v = sc[:nc]
        bufs = dict(zip(buf_keys, sc[nc:nc + len(buf_keys)]))
        sems = sc[nc + len(buf_keys):]
        gsem = dict(zip(buf_keys, sems[:len(buf_keys)]))
        ssem = dict(zip(buf_keys, sems[len(buf_keys):]))

        wid = lax.axis_index("s") * _NC + lax.axis_index("c")
        b = wid // 4
        blk = wid % 4
        t0 = 4 * blk
        slab0 = pl.multiple_of((16 * b + t0) * _N, _SLAB)
        slab = x_hbm.at[pl.ds(slab0, _SLAB)]

        idx_copies = [pltpu.async_copy(src, dst, gsem["C"])
                      for src, dst in zip(idx_in, idx_v)]
        for h in idx_copies:
            h.wait()

        units = []
        for j in range(4):
            for u, (k0, ln) in enumerate(_CHUNKS):
                key = "C" if ln == 4 else f"A{len(units) % 4}"

                def g_u(key=key, j=j, u=u, ln=ln):
                    dst = (bufs[key] if ln in (4, 40)
                           else bufs[key].at[pl.ds(0, ln)])
                    return pltpu.async_copy(
                        slab.at[idx_v[u].at[j]], dst, gsem[key])

                def s_u(key=key, j=j, k0=k0, ln=ln):
                    src = (bufs[key] if ln in (4, 40)
                           else bufs[key].at[pl.ds(0, ln)])
                    return pltpu.async_copy(
                        src, out_hbm.at[b, t0 + j, pl.ds(k0, ln), :],
                        ssem[key])

                units.append((key, g_u, s_u))

        last_store = {}
        gathers = [None] * len(units)

        def issue(i):
            key = units[i][0]
            h = last_store.pop(key, None)
            if h is not None:
                h.wait()
            gathers[i] = units[i][1]()

        la = 3
        for i in range(la):
            issue(i)
        for i, (key, _, store) in enumerate(units):
            if i + la < len(units):
                issue(i + la)
            gathers[i].wait()
            last_store[key] = store()
        for h in last_store.values():
            h.wait()

    return _tube_gather


def kernel(x):
    args = [jnp.asarray(a) for a in _CHUNK_IDX]
    return _build_tube_gather()(x.reshape(_ROWS, _D), *args)
